# baseline (speedup 1.0000x reference)
import jax
import jax.numpy as jnp
from jax.experimental import pallas as pl
from jax.experimental.pallas import tpu as pltpu

_BM = 2048


def _gate_matmul(x_ref, wgt_ref, o_ref):
    o_ref[...] = jnp.dot(x_ref[...], wgt_ref[...],
                         preferred_element_type=jnp.float32)


def kernel(x, wg):
    m, k = x.shape
    e = wg.shape[0]
    wgt = wg.T
    return pl.pallas_call(
        _gate_matmul,
        grid=(m // _BM,),
        in_specs=[
            pl.BlockSpec((_BM, k), lambda i: (i, 0)),
            pl.BlockSpec((k, e), lambda i: (0, 0)),
        ],
        out_specs=pl.BlockSpec((_BM, e), lambda i: (i, 0)),
        out_shape=jax.ShapeDtypeStruct((m, e), jnp.float32),
    )(x, wgt)


# speedup vs baseline: 1.0150x; 1.0150x over previous
import jax
import jax.numpy as jnp
from jax.experimental import pallas as pl
from jax.experimental.pallas import tpu as pltpu

_BM = 1024
_NBUF = 8


def _copy(x_hbm, xbuf, sems, block, slot):
    return pltpu.make_async_copy(
        x_hbm.at[pl.ds(block * _BM, _BM), :], xbuf.at[slot], sems.at[slot])


def _gate_matmul(x_hbm, wgt_ref, o_ref, xbuf, sems):
    i = pl.program_id(0)
    nsteps = pl.num_programs(0)

    @pl.when(i == 0)
    def _warmup():
        for b in range(_NBUF):
            _copy(x_hbm, xbuf, sems, b, b).start()

    slot = jax.lax.rem(i, _NBUF)
    _copy(x_hbm, xbuf, sems, i, slot).wait()
    o_ref[...] = jnp.dot(xbuf[slot], wgt_ref[...],
                         preferred_element_type=jnp.float32)

    nxt = i + _NBUF

    @pl.when(nxt < nsteps)
    def _prefetch():
        _copy(x_hbm, xbuf, sems, nxt, slot).start()


def kernel(x, wg):
    m, k = x.shape
    e = wg.shape[0]
    wgt = wg.T
    return pl.pallas_call(
        _gate_matmul,
        grid=(m // _BM,),
        in_specs=[
            pl.BlockSpec(memory_space=pl.ANY),
            pl.BlockSpec((k, e), lambda i: (0, 0)),
        ],
        out_specs=pl.BlockSpec((_BM, e), lambda i: (i, 0)),
        out_shape=jax.ShapeDtypeStruct((m, e), jnp.float32),
        scratch_shapes=[
            pltpu.VMEM((_NBUF, _BM, k), jnp.float32),
            pltpu.SemaphoreType.DMA((_NBUF,)),
        ],
    )(x, wgt)


# speedup vs baseline: 1.0204x; 1.0054x over previous
import jax
import jax.numpy as jnp
from jax.experimental import pallas as pl
from jax.experimental.pallas import tpu as pltpu

_BM = 1024
_NBUF = 8


def _copy(x_hbm, xbuf, sems, block, slot):
    return pltpu.make_async_copy(
        x_hbm.at[pl.ds(block * _BM, _BM), :], xbuf.at[slot], sems.at[slot])


def _gate_matmul(x_hbm, wgt_ref, o_ref, xbuf, sems):
    i = pl.program_id(0)
    nsteps = pl.num_programs(0)

    @pl.when(i == 0)
    def _warmup():
        for b in range(_NBUF):
            _copy(x_hbm, xbuf, sems, b, b).start()

    slot = jax.lax.rem(i, _NBUF)
    _copy(x_hbm, xbuf, sems, i, slot).wait()
    o_ref[...] = jnp.dot(xbuf[slot].astype(jnp.bfloat16),
                         wgt_ref[...].astype(jnp.bfloat16),
                         preferred_element_type=jnp.float32)

    nxt = i + _NBUF

    @pl.when(nxt < nsteps)
    def _prefetch():
        _copy(x_hbm, xbuf, sems, nxt, slot).start()


def kernel(x, wg):
    m, k = x.shape
    e = wg.shape[0]
    wgt = wg.T
    return pl.pallas_call(
        _gate_matmul,
        grid=(m // _BM,),
        in_specs=[
            pl.BlockSpec(memory_space=pl.ANY),
            pl.BlockSpec((k, e), lambda i: (0, 0)),
        ],
        out_specs=pl.BlockSpec((_BM, e), lambda i: (i, 0)),
        out_shape=jax.ShapeDtypeStruct((m, e), jnp.float32),
        scratch_shapes=[
            pltpu.VMEM((_NBUF, _BM, k), jnp.float32),
            pltpu.SemaphoreType.DMA((_NBUF,)),
        ],
    )(x, wgt)
